# baseline (speedup 1.0000x reference)
import functools

import jax
import jax.numpy as jnp
from jax import lax
from jax.experimental import pallas as pl
from jax.experimental.pallas import tpu as pltpu
from jax.experimental.pallas import tpu_sc as plsc

K_PROP = 10
ALPHA = 0.1
NS = 16
NC = 2
CH = 128
KB = 3
RB = 128
BN = 2000


def _mlp_body(x_ref, w1_ref, b1_ref, w2_ref, b2_ref, o_ref):
    h1 = jnp.dot(x_ref[...], w1_ref[...], preferred_element_type=jnp.float32)
    h1 = jnp.maximum(h1 + b1_ref[...], 0.0)
    o_ref[...] = (
        jnp.dot(h1, w2_ref[...], preferred_element_type=jnp.float32)
        + b2_ref[...]
    )


def _prep_body(h_ref, d_ref, u0_ref, c32_ref, g_ref, sq_ref):
    deg = d_ref[:, 0:1] + 1.0
    dinv = lax.rsqrt(deg)
    h = h_ref[...]
    u0_ref[...] = dinv * h
    c32_ref[...] = jnp.broadcast_to((1.0 - ALPHA) * dinv * dinv, h.shape)
    g_ref[...] = (ALPHA * dinv) * h
    sq_ref[...] = jnp.broadcast_to(jnp.sqrt(deg), d_ref.shape)


def _final_body(u_ref, sq_ref, o_ref):
    z = u_ref[...] * sq_ref[:, 0:1]
    m = jnp.max(z, axis=1, keepdims=True)
    zs = z - m
    o_ref[...] = zs - jnp.log(jnp.sum(jnp.exp(zs), axis=1, keepdims=True))


def _compute_offsets(idxv, offv, base, half, trash):
    for b in range(KB):
        for k in range(CH // 16):
            sl = pl.ds(16 * k, 16)
            cv = idxv[0, KB + b, sl]
            off = cv - base
            ok = (off >= 0) & (off < half)
            offv[0, b, sl] = jnp.where(ok, off, trash)


def _make_deg_kernel(n, half, aggr, gpt):
    trash = half
    hpt = (-(-half // NS) + 7) // 8 * 8
    nch = -(-hpt // RB)
    mesh = plsc.VectorSubcoreMesh(core_axis_name="c", subcore_axis_name="s")

    @functools.partial(
        pl.kernel,
        out_type=jax.ShapeDtypeStruct((n, 16), jnp.float32),
        mesh=mesh,
        compiler_params=pltpu.CompilerParams(use_tc_tiling_on_sc=False),
        scratch_types=[
            pltpu.VMEM_SHARED((aggr, 16), jnp.float32),
            pltpu.VMEM((RB, 16), jnp.float32),
            pltpu.VMEM((1, 2 * KB, CH), jnp.int32),
            pltpu.VMEM((1, KB, CH), jnp.int32),
            pltpu.VMEM((RB, 16), jnp.float32),
        ],
    )
    def deg_kernel(eidx_hbm, deg_hbm, agg, fill, idxv, offv, stage):
        c = lax.axis_index("c")
        s = lax.axis_index("s")
        base = c * half

        def fill_rows(val):
            def frow(r, _):
                fill[r, pl.ds(0, 16)] = jnp.full((16,), val, jnp.float32)
                return 0
            lax.fori_loop(0, RB, frow, 0)

        fill_rows(0.0)

        def zchunk(j, _):
            lo = jnp.minimum(s * hpt + j * RB, aggr - RB)
            pltpu.sync_copy(fill, agg.at[pl.ds(lo, RB)])
            return 0
        lax.fori_loop(0, nch + 1, zchunk, 0)
        fill_rows(1.0)
        plsc.subcore_barrier()

        def group(g, _):
            pltpu.sync_copy(eidx_hbm.at[pl.ds(s * gpt + g, 1)], idxv)
            _compute_offsets(idxv, offv, base, half, trash)
            for b in range(KB):
                pltpu.sync_copy(fill, agg.at[offv.at[0, b]], add=True)
            return 0
        lax.fori_loop(0, gpt, group, 0)
        plsc.subcore_barrier()

        def out_chunk(j, _):
            r = jnp.minimum(s * hpt + j * RB, half - RB)
            pltpu.sync_copy(agg.at[pl.ds(r, RB)], stage)
            pltpu.sync_copy(stage, deg_hbm.at[pl.ds(base + r, RB)])
            return 0
        lax.fori_loop(0, nch, out_chunk, 0)

    return deg_kernel


def _make_prop_kernel(n, half, aggr, gpt):
    trash = half
    hpt = (-(-half // NS) + 7) // 8 * 8
    nch = -(-hpt // RB)
    n_pairs = gpt // 2 - 1
    mesh = plsc.VectorSubcoreMesh(core_axis_name="c", subcore_axis_name="s")

    @functools.partial(
        pl.kernel,
        out_type=jax.ShapeDtypeStruct((n, 32), jnp.float32),
        mesh=mesh,
        compiler_params=pltpu.CompilerParams(use_tc_tiling_on_sc=False),
        scratch_types=[
            pltpu.VMEM_SHARED((aggr, 32), jnp.float32),
            pltpu.VMEM((1, 2 * KB, CH), jnp.int32),
            pltpu.VMEM((1, 2 * KB, CH), jnp.int32),
            pltpu.VMEM((1, KB, CH), jnp.int32),
            pltpu.VMEM((1, KB, CH), jnp.int32),
            pltpu.VMEM((KB, CH, 32), jnp.float32),
            pltpu.VMEM((KB, CH, 32), jnp.float32),
            pltpu.SemaphoreType.DMA,
            pltpu.SemaphoreType.DMA,
            pltpu.SemaphoreType.DMA,
            pltpu.SemaphoreType.DMA,
        ],
    )
    def prop_kernel(u_hbm, eidx_hbm, c32_hbm, g_hbm, out_hbm,
                    agg, idxv0, idxv1, offv0, offv1,
                    rows0, rows1, gsem0, gsem1, ssem0, ssem1):
        zbuf = rows0.at[0]
        bagg, bu = rows0.at[1], rows0.at[2]
        bc, bg = rows1.at[0], rows1.at[1]
        c = lax.axis_index("c")
        s = lax.axis_index("s")
        base = c * half

        def zrow(r, _):
            zbuf[r, pl.ds(0, 16)] = jnp.zeros((16,), jnp.float32)
            zbuf[r, pl.ds(16, 16)] = jnp.zeros((16,), jnp.float32)
            return 0
        lax.fori_loop(0, RB, zrow, 0)
        zc = zbuf

        def zchunk(j, _):
            lo = jnp.minimum(s * hpt + j * RB, aggr - RB)
            pltpu.sync_copy(zc, agg.at[pl.ds(lo, RB)])
            return 0
        lax.fori_loop(0, nch + 1, zchunk, 0)
        plsc.subcore_barrier()

        def fire(g, idxv, offv, rows, gsem):
            gg = s * gpt + g
            pltpu.sync_copy(eidx_hbm.at[pl.ds(gg, 1)], idxv)
            for b in range(KB):
                pltpu.async_copy(u_hbm.at[idxv.at[0, b]], rows.at[b], gsem)
            _compute_offsets(idxv, offv, base, half, trash)

        def drain(idxv, offv, rows, gsem):
            for b in range(KB):
                pltpu.make_async_copy(
                    u_hbm.at[idxv.at[0, b]], rows.at[b], gsem).wait()
            for b in range(KB):
                pltpu.sync_copy(rows.at[b], agg.at[offv.at[0, b]], add=True)

        fire(0, idxv0, offv0, rows0, gsem0)
        fire(1, idxv1, offv1, rows1, gsem1)

        def pair(j, _):
            drain(idxv0, offv0, rows0, gsem0)
            fire(2 * j + 2, idxv0, offv0, rows0, gsem0)
            drain(idxv1, offv1, rows1, gsem1)
            fire(2 * j + 3, idxv1, offv1, rows1, gsem1)
            return 0
        lax.fori_loop(0, n_pairs, pair, 0)
        drain(idxv0, offv0, rows0, gsem0)
        drain(idxv1, offv1, rows1, gsem1)
        plsc.subcore_barrier()

        def bchunk(j, _):
            r = jnp.minimum(s * hpt + j * RB, half - RB)
            lo = base + r
            pltpu.sync_copy(agg.at[pl.ds(r, RB)], bagg)
            pltpu.sync_copy(u_hbm.at[pl.ds(lo, RB)], bu)
            pltpu.sync_copy(c32_hbm.at[pl.ds(lo, RB)], bc)
            pltpu.sync_copy(g_hbm.at[pl.ds(lo, RB)], bg)

            def rrow(rr, _):
                for k in range(2):
                    sl = pl.ds(16 * k, 16)
                    bagg[rr, sl] = bc[rr, sl] * (bagg[rr, sl] + bu[rr, sl]) \
                        + bg[rr, sl]
                return 0
            lax.fori_loop(0, RB, rrow, 0)
            pltpu.sync_copy(bagg, out_hbm.at[pl.ds(lo, RB)])
            return 0
        lax.fori_loop(0, nch, bchunk, 0)

    return prop_kernel


def kernel(x, edge_index, W1, b1, W2, b2):
    n, ic = x.shape
    hc = W1.shape[1]
    oc = W2.shape[1]
    e = edge_index.shape[1]

    half = n // 2
    aggr = half + NS
    slab = KB * CH
    group_edges = NS * slab * 2
    ep = ((e + group_edges - 1) // group_edges) * group_edges
    nbs = ep // slab
    gpt = nbs // NS

    row = edge_index[0].astype(jnp.int32)
    col = edge_index[1].astype(jnp.int32)
    pad = ep - e
    row3 = jnp.concatenate(
        [row, jnp.zeros((pad,), jnp.int32)]).reshape(nbs, KB, CH)
    col3 = jnp.concatenate(
        [col, jnp.full((pad,), n, jnp.int32)]).reshape(nbs, KB, CH)
    eidx = jnp.concatenate([row3, col3], axis=1)

    grid = (n // BN,)
    h = pl.pallas_call(
        _mlp_body,
        grid=grid,
        in_specs=[
            pl.BlockSpec((BN, ic), lambda i: (i, 0)),
            pl.BlockSpec((ic, hc), lambda i: (0, 0)),
            pl.BlockSpec((1, hc), lambda i: (0, 0)),
            pl.BlockSpec((hc, oc), lambda i: (0, 0)),
            pl.BlockSpec((1, oc), lambda i: (0, 0)),
        ],
        out_specs=pl.BlockSpec((BN, oc), lambda i: (i, 0)),
        out_shape=jax.ShapeDtypeStruct((n, oc), jnp.float32),
    )(x, W1, b1.reshape(1, hc), W2, b2.reshape(1, oc))

    deg16 = _make_deg_kernel(n, half, aggr, gpt)(eidx)

    u, c32, g, sq = pl.pallas_call(
        _prep_body,
        grid=grid,
        in_specs=[
            pl.BlockSpec((BN, oc), lambda i: (i, 0)),
            pl.BlockSpec((BN, 16), lambda i: (i, 0)),
        ],
        out_specs=[
            pl.BlockSpec((BN, oc), lambda i: (i, 0)),
            pl.BlockSpec((BN, oc), lambda i: (i, 0)),
            pl.BlockSpec((BN, oc), lambda i: (i, 0)),
            pl.BlockSpec((BN, 16), lambda i: (i, 0)),
        ],
        out_shape=[
            jax.ShapeDtypeStruct((n, oc), jnp.float32),
            jax.ShapeDtypeStruct((n, oc), jnp.float32),
            jax.ShapeDtypeStruct((n, oc), jnp.float32),
            jax.ShapeDtypeStruct((n, 16), jnp.float32),
        ],
    )(h, deg16)

    prop = _make_prop_kernel(n, half, aggr, gpt)
    for _ in range(K_PROP):
        u = prop(u, eidx, c32, g)

    out = pl.pallas_call(
        _final_body,
        grid=grid,
        in_specs=[
            pl.BlockSpec((BN, oc), lambda i: (i, 0)),
            pl.BlockSpec((BN, 16), lambda i: (i, 0)),
        ],
        out_specs=pl.BlockSpec((BN, oc), lambda i: (i, 0)),
        out_shape=jax.ShapeDtypeStruct((n, oc), jnp.float32),
    )(u, sq)
    return out


# speedup vs baseline: 2.7426x; 2.7426x over previous
import functools

import jax
import jax.numpy as jnp
from jax import lax
from jax.experimental import pallas as pl
from jax.experimental.pallas import tpu as pltpu
from jax.experimental.pallas import tpu_sc as plsc

K_PROP = 10
ALPHA = 0.1
NS = 16
NC = 2
CH = 128
KB = 3
RB = 128
BN = 2000


def _mlp_body(x_ref, w1_ref, b1_ref, w2_ref, b2_ref, o_ref):
    h1 = jnp.dot(x_ref[...], w1_ref[...], preferred_element_type=jnp.float32)
    h1 = jnp.maximum(h1 + b1_ref[...], 0.0)
    o_ref[...] = (
        jnp.dot(h1, w2_ref[...], preferred_element_type=jnp.float32)
        + b2_ref[...]
    )


def _prep_body(h_ref, d_ref, u0_ref, c32_ref, g_ref, sq_ref):
    deg = d_ref[:, 0:1] + 1.0
    dinv = lax.rsqrt(deg)
    h = h_ref[...]
    u0_ref[...] = dinv * h
    c32_ref[...] = jnp.broadcast_to((1.0 - ALPHA) * dinv * dinv, h.shape)
    g_ref[...] = (ALPHA * dinv) * h
    sq_ref[...] = jnp.broadcast_to(jnp.sqrt(deg), d_ref.shape)


def _final_body(u_ref, sq_ref, o_ref):
    z = u_ref[...] * sq_ref[:, 0:1]
    m = jnp.max(z, axis=1, keepdims=True)
    zs = z - m
    o_ref[...] = zs - jnp.log(jnp.sum(jnp.exp(zs), axis=1, keepdims=True))


def _compute_offsets(idxv, offv, base, half, trash):
    for b in range(KB):
        for k in range(CH // 16):
            sl = pl.ds(16 * k, 16)
            cv = idxv[0, KB + b, sl]
            off = cv - base
            ok = (off >= 0) & (off < half)
            offv[0, b, sl] = jnp.where(ok, off, trash)


def _make_part_kernel(n, half, gpt, capc):
    trash = half
    stg = 672
    mesh = plsc.VectorSubcoreMesh(core_axis_name="c", subcore_axis_name="s")

    @functools.partial(
        pl.kernel,
        out_type=(
            jax.ShapeDtypeStruct((NC * NS * capc * 2 * CH,), jnp.int32),
            jax.ShapeDtypeStruct((NC * NS * 16,), jnp.int32),
        ),
        mesh=mesh,
        compiler_params=pltpu.CompilerParams(
            use_tc_tiling_on_sc=False, needs_layout_passes=False),
        scratch_types=[
            pltpu.VMEM((1, 2 * KB, CH), jnp.int32),
            pltpu.VMEM((stg,), jnp.int32),
            pltpu.VMEM((stg,), jnp.int32),
            pltpu.VMEM((16,), jnp.int32),
        ],
    )
    def part_kernel(eidx_hbm, part_hbm, cnt_hbm, idxv, srow, soff, cbuf):
        c = lax.axis_index("c")
        s = lax.axis_index("s")
        base = c * half
        t = c * NS + s
        tbase = t * capc

        def emit(cur_ck):
            cur, ck = cur_ck
            o = (tbase + ck) * 2 * CH
            pltpu.sync_copy(srow.at[pl.ds(0, CH)],
                            part_hbm.at[pl.ds(o, CH)])
            pltpu.sync_copy(soff.at[pl.ds(0, CH)],
                            part_hbm.at[pl.ds(o + CH, CH)])
            for i in range((stg - CH) // 16):
                d = pl.ds(i * 16, 16)
                sr = pl.ds(CH + i * 16, 16)
                srow[d] = srow[sr]
                soff[d] = soff[sr]
            return (cur - CH, ck + 1)

        def keep(cur_ck):
            return cur_ck

        lane = lax.iota(jnp.int32, 16)

        def slab(g, cur_ck):
            pltpu.sync_copy(eidx_hbm.at[pl.ds(s * gpt + g, 1)], idxv)
            cur, ck = cur_ck
            for b in range(KB):
                for k in range(CH // 16):
                    sl = pl.ds(16 * k, 16)
                    rv = idxv[0, b, sl]
                    cv = idxv[0, KB + b, sl]
                    off = cv - base
                    m = (off >= 0) & (off < half)
                    key = jnp.where(m, lane, 16 + lane)
                    _, rv_s = plsc.sort_key_val(key, rv)
                    _, off_s = plsc.sort_key_val(key, off)
                    srow[pl.ds(cur, 16)] = rv_s
                    soff[pl.ds(cur, 16)] = off_s
                    cur = cur + jnp.sum(m.astype(jnp.int32))
            for _ in range(3):
                cur, ck = lax.cond(cur >= CH, emit, keep, (cur, ck))
            return (cur, ck)

        cur, ck = lax.fori_loop(0, gpt, slab,
                                (jnp.int32(0), jnp.int32(0)))

        for i in range(CH // 16):
            srow[pl.ds(cur + i * 16, 16)] = jnp.zeros((16,), jnp.int32)
            soff[pl.ds(cur + i * 16, 16)] = jnp.full((16,), trash, jnp.int32)

        def emit_if(cur_ck):
            return lax.cond(cur_ck[0] > 0, emit, keep, cur_ck)
        for _ in range(4):
            cur, ck = emit_if((cur, ck))

        for i in range(CH // 16):
            srow[pl.ds(i * 16, 16)] = jnp.zeros((16,), jnp.int32)
            soff[pl.ds(i * 16, 16)] = jnp.full((16,), trash, jnp.int32)

        def emit_pad(ck_):
            o = (tbase + ck_) * 2 * CH
            pltpu.sync_copy(srow.at[pl.ds(0, CH)],
                            part_hbm.at[pl.ds(o, CH)])
            pltpu.sync_copy(soff.at[pl.ds(0, CH)],
                            part_hbm.at[pl.ds(o + CH, CH)])
            return ck_ + 1

        def keep1(ck_):
            return ck_
        for _ in range(6):
            ck = lax.cond((ck < 6) | (ck % 6 != 0), emit_pad, keep1, ck)

        lane = lax.iota(jnp.int32, 16)
        cbuf[pl.ds(0, 16)] = jnp.where(lane == 0, ck, 0)
        pltpu.sync_copy(cbuf, cnt_hbm.at[pl.ds(t * 16, 16)])

    return part_kernel


def _make_deg_kernel(n, half, aggr, capc):
    hpt = (-(-half // NS) + 7) // 8 * 8
    nch = -(-hpt // RB)
    mesh = plsc.VectorSubcoreMesh(core_axis_name="c", subcore_axis_name="s")

    @functools.partial(
        pl.kernel,
        out_type=jax.ShapeDtypeStruct((n, 16), jnp.float32),
        mesh=mesh,
        compiler_params=pltpu.CompilerParams(
            use_tc_tiling_on_sc=False, needs_layout_passes=False),
        scratch_types=[
            pltpu.VMEM_SHARED((aggr, 16), jnp.float32),
            pltpu.VMEM((RB, 16), jnp.float32),
            pltpu.VMEM((3, 2, CH), jnp.int32),
            pltpu.VMEM((16,), jnp.int32),
            pltpu.VMEM((RB, 16), jnp.float32),
        ],
    )
    def deg_kernel(part_hbm, cnt_hbm, deg_hbm, agg, fill, idxv, cbuf, stage):
        c = lax.axis_index("c")
        s = lax.axis_index("s")
        base = c * half
        t = c * NS + s
        tbase = t * capc

        def fill_rows(val):
            def frow(r, _):
                fill[r, pl.ds(0, 16)] = jnp.full((16,), val, jnp.float32)
                return 0
            lax.fori_loop(0, RB, frow, 0)

        fill_rows(0.0)

        def zchunk(j, _):
            lo = jnp.minimum(s * hpt + j * RB, aggr - RB)
            pltpu.sync_copy(fill, agg.at[pl.ds(lo, RB)])
            return 0
        lax.fori_loop(0, nch + 1, zchunk, 0)
        fill_rows(1.0)
        pltpu.sync_copy(cnt_hbm.at[pl.ds(t * 16, 16)], cbuf)
        nck = cbuf[pl.ds(0, 16)][0]
        plsc.subcore_barrier()

        def group(g, _):
            pltpu.sync_copy(part_hbm.at[pl.ds(tbase + g * 3, 3)], idxv)
            for b in range(3):
                pltpu.sync_copy(fill, agg.at[idxv.at[b, 1]], add=True)
            return 0
        lax.fori_loop(0, nck // 3, group, 0)
        plsc.subcore_barrier()

        def out_chunk(j, _):
            r = jnp.minimum(s * hpt + j * RB, half - RB)
            pltpu.sync_copy(agg.at[pl.ds(r, RB)], stage)
            pltpu.sync_copy(stage, deg_hbm.at[pl.ds(base + r, RB)])
            return 0
        lax.fori_loop(0, nch, out_chunk, 0)

    return deg_kernel


def _make_prop_kernel(n, half, aggr, capc):
    hpt = (-(-half // NS) + 7) // 8 * 8
    nch = -(-hpt // RB)
    mesh = plsc.VectorSubcoreMesh(core_axis_name="c", subcore_axis_name="s")

    @functools.partial(
        pl.kernel,
        out_type=jax.ShapeDtypeStruct((n, 32), jnp.float32),
        mesh=mesh,
        compiler_params=pltpu.CompilerParams(
            use_tc_tiling_on_sc=False, needs_layout_passes=False),
        scratch_types=[
            pltpu.VMEM_SHARED((aggr, 32), jnp.float32),
            pltpu.VMEM((3, 2, CH), jnp.int32),
            pltpu.VMEM((3, 2, CH), jnp.int32),
            pltpu.VMEM((3, CH, 32), jnp.float32),
            pltpu.VMEM((3, CH, 32), jnp.float32),
            pltpu.VMEM((16,), jnp.int32),
            pltpu.SemaphoreType.DMA,
            pltpu.SemaphoreType.DMA,
        ],
    )
    def prop_kernel(u_hbm, part_hbm, cnt_hbm, c32_hbm, g_hbm, out_hbm,
                    agg, idxv0, idxv1, rows0, rows1, cbuf, gsem0, gsem1):
        c = lax.axis_index("c")
        s = lax.axis_index("s")
        base = c * half
        t = c * NS + s
        tbase = t * capc

        zbuf = rows0.at[0]
        bagg, bu = rows0.at[1], rows0.at[2]
        bc, bg = rows1.at[0], rows1.at[1]

        def zrow(r, _):
            zbuf[r, pl.ds(0, 16)] = jnp.zeros((16,), jnp.float32)
            zbuf[r, pl.ds(16, 16)] = jnp.zeros((16,), jnp.float32)
            return 0
        lax.fori_loop(0, RB, zrow, 0)

        def zchunk(j, _):
            lo = jnp.minimum(s * hpt + j * RB, aggr - RB)
            pltpu.sync_copy(zbuf, agg.at[pl.ds(lo, RB)])
            return 0
        lax.fori_loop(0, nch + 1, zchunk, 0)
        pltpu.sync_copy(cnt_hbm.at[pl.ds(t * 16, 16)], cbuf)
        nck = cbuf[pl.ds(0, 16)][0]
        plsc.subcore_barrier()

        def fire(gi, idxv, rows, gsem):
            pltpu.sync_copy(part_hbm.at[pl.ds(tbase + gi * 3, 3)], idxv)
            for b in range(3):
                pltpu.async_copy(u_hbm.at[idxv.at[b, 0]], rows.at[b], gsem)

        def drain(idxv, rows, gsem):
            for b in range(3):
                pltpu.make_async_copy(
                    u_hbm.at[idxv.at[b, 0]], rows.at[b], gsem).wait()
            for b in range(3):
                pltpu.sync_copy(rows.at[b], agg.at[idxv.at[b, 1]], add=True)

        fire(0, idxv0, rows0, gsem0)
        fire(1, idxv1, rows1, gsem1)

        def pair(j, _):
            drain(idxv0, rows0, gsem0)
            fire(2 * j + 2, idxv0, rows0, gsem0)
            drain(idxv1, rows1, gsem1)
            fire(2 * j + 3, idxv1, rows1, gsem1)
            return 0
        lax.fori_loop(0, nck // 6 - 1, pair, 0)
        drain(idxv0, rows0, gsem0)
        drain(idxv1, rows1, gsem1)
        plsc.subcore_barrier()

        def bchunk(j, _):
            r = jnp.minimum(s * hpt + j * RB, half - RB)
            lo = base + r
            pltpu.sync_copy(agg.at[pl.ds(r, RB)], bagg)
            pltpu.sync_copy(u_hbm.at[pl.ds(lo, RB)], bu)
            pltpu.sync_copy(c32_hbm.at[pl.ds(lo, RB)], bc)
            pltpu.sync_copy(g_hbm.at[pl.ds(lo, RB)], bg)

            def rrow(rr, _):
                for k in range(2):
                    sl = pl.ds(16 * k, 16)
                    bagg[rr, sl] = bc[rr, sl] * (bagg[rr, sl] + bu[rr, sl]) \
                        + bg[rr, sl]
                return 0
            lax.fori_loop(0, RB, rrow, 0)
            pltpu.sync_copy(bagg, out_hbm.at[pl.ds(lo, RB)])
            return 0
        lax.fori_loop(0, nch, bchunk, 0)

    return prop_kernel


def kernel(x, edge_index, W1, b1, W2, b2):
    n, ic = x.shape
    hc = W1.shape[1]
    oc = W2.shape[1]
    e = edge_index.shape[1]

    half = n // 2
    aggr = half + NS
    slab = KB * CH
    group_edges = NS * slab * 2
    ep = ((e + group_edges - 1) // group_edges) * group_edges
    nbs = ep // slab
    gpt = nbs // NS

    row = edge_index[0].astype(jnp.int32)
    col = edge_index[1].astype(jnp.int32)
    pad = ep - e
    row3 = jnp.concatenate(
        [row, jnp.zeros((pad,), jnp.int32)]).reshape(nbs, KB, CH)
    col3 = jnp.concatenate(
        [col, jnp.full((pad,), n, jnp.int32)]).reshape(nbs, KB, CH)
    eidx = jnp.concatenate([row3, col3], axis=1)

    grid = (n // BN,)
    h = pl.pallas_call(
        _mlp_body,
        grid=grid,
        in_specs=[
            pl.BlockSpec((BN, ic), lambda i: (i, 0)),
            pl.BlockSpec((ic, hc), lambda i: (0, 0)),
            pl.BlockSpec((1, hc), lambda i: (0, 0)),
            pl.BlockSpec((hc, oc), lambda i: (0, 0)),
            pl.BlockSpec((1, oc), lambda i: (0, 0)),
        ],
        out_specs=pl.BlockSpec((BN, oc), lambda i: (i, 0)),
        out_shape=jax.ShapeDtypeStruct((n, oc), jnp.float32),
    )(x, W1, b1.reshape(1, hc), W2, b2.reshape(1, oc))

    capc = (gpt * KB // 6 + 1) * 6
    part1, cnt1 = _make_part_kernel(n, half, gpt, capc)(eidx)
    part3 = part1.reshape(NC * NS * capc, 2, CH)

    deg16 = _make_deg_kernel(n, half, aggr, capc)(part3, cnt1)

    u, c32, g, sq = pl.pallas_call(
        _prep_body,
        grid=grid,
        in_specs=[
            pl.BlockSpec((BN, oc), lambda i: (i, 0)),
            pl.BlockSpec((BN, 16), lambda i: (i, 0)),
        ],
        out_specs=[
            pl.BlockSpec((BN, oc), lambda i: (i, 0)),
            pl.BlockSpec((BN, oc), lambda i: (i, 0)),
            pl.BlockSpec((BN, oc), lambda i: (i, 0)),
            pl.BlockSpec((BN, 16), lambda i: (i, 0)),
        ],
        out_shape=[
            jax.ShapeDtypeStruct((n, oc), jnp.float32),
            jax.ShapeDtypeStruct((n, oc), jnp.float32),
            jax.ShapeDtypeStruct((n, oc), jnp.float32),
            jax.ShapeDtypeStruct((n, 16), jnp.float32),
        ],
    )(h, deg16)

    prop = _make_prop_kernel(n, half, aggr, capc)
    for _ in range(K_PROP):
        u = prop(u, part3, cnt1, c32, g)

    out = pl.pallas_call(
        _final_body,
        grid=grid,
        in_specs=[
            pl.BlockSpec((BN, oc), lambda i: (i, 0)),
            pl.BlockSpec((BN, 16), lambda i: (i, 0)),
        ],
        out_specs=pl.BlockSpec((BN, oc), lambda i: (i, 0)),
        out_shape=jax.ShapeDtypeStruct((n, oc), jnp.float32),
    )(u, sq)
    return out


# speedup vs baseline: 2.7686x; 1.0095x over previous
import functools

import jax
import jax.numpy as jnp
from jax import lax
from jax.experimental import pallas as pl
from jax.experimental.pallas import tpu as pltpu
from jax.experimental.pallas import tpu_sc as plsc

K_PROP = 10
ALPHA = 0.1
NS = 16
NC = 2
CH = 128
KB = 3
RB = 128
BN = 2000


def _mlp_body(x_ref, w1_ref, b1_ref, w2_ref, b2_ref, o_ref):
    h1 = jnp.dot(x_ref[...], w1_ref[...], preferred_element_type=jnp.float32)
    h1 = jnp.maximum(h1 + b1_ref[...], 0.0)
    o_ref[...] = (
        jnp.dot(h1, w2_ref[...], preferred_element_type=jnp.float32)
        + b2_ref[...]
    )


def _prep_body(h_ref, d_ref, u0_ref, c32_ref, g_ref, sq_ref):
    deg = d_ref[:, 0:1] + 1.0
    dinv = lax.rsqrt(deg)
    h = h_ref[...]
    u0_ref[...] = dinv * h
    c32_ref[...] = jnp.broadcast_to((1.0 - ALPHA) * dinv * dinv, h.shape)
    g_ref[...] = (ALPHA * dinv) * h
    sq_ref[...] = jnp.broadcast_to(jnp.sqrt(deg), d_ref.shape)


def _final_body(u_ref, sq_ref, o_ref):
    z = u_ref[...] * sq_ref[:, 0:1]
    m = jnp.max(z, axis=1, keepdims=True)
    zs = z - m
    o_ref[...] = zs - jnp.log(jnp.sum(jnp.exp(zs), axis=1, keepdims=True))


def _compute_offsets(idxv, offv, base, half, trash):
    for b in range(KB):
        for k in range(CH // 16):
            sl = pl.ds(16 * k, 16)
            cv = idxv[0, KB + b, sl]
            off = cv - base
            ok = (off >= 0) & (off < half)
            offv[0, b, sl] = jnp.where(ok, off, trash)


def _make_part_kernel(n, half, gpt, capc):
    trash = half
    stg = 672
    mesh = plsc.VectorSubcoreMesh(core_axis_name="c", subcore_axis_name="s")

    @functools.partial(
        pl.kernel,
        out_type=(
            jax.ShapeDtypeStruct((NC * NS * capc * 2 * CH,), jnp.int32),
            jax.ShapeDtypeStruct((NC * NS * 16,), jnp.int32),
        ),
        mesh=mesh,
        compiler_params=pltpu.CompilerParams(
            use_tc_tiling_on_sc=False, needs_layout_passes=False),
        scratch_types=[
            pltpu.VMEM((1, 2 * KB, CH), jnp.int32),
            pltpu.VMEM((stg,), jnp.int32),
            pltpu.VMEM((stg,), jnp.int32),
            pltpu.VMEM((16,), jnp.int32),
        ],
    )
    def part_kernel(eidx_hbm, part_hbm, cnt_hbm, idxv, srow, soff, cbuf):
        c = lax.axis_index("c")
        s = lax.axis_index("s")
        base = c * half
        t = c * NS + s
        tbase = t * capc

        def emit(cur_ck):
            cur, ck = cur_ck
            o = (tbase + ck) * 2 * CH
            pltpu.sync_copy(srow.at[pl.ds(0, CH)],
                            part_hbm.at[pl.ds(o, CH)])
            pltpu.sync_copy(soff.at[pl.ds(0, CH)],
                            part_hbm.at[pl.ds(o + CH, CH)])
            for i in range((stg - CH) // 16):
                d = pl.ds(i * 16, 16)
                sr = pl.ds(CH + i * 16, 16)
                srow[d] = srow[sr]
                soff[d] = soff[sr]
            return (cur - CH, ck + 1)

        def keep(cur_ck):
            return cur_ck

        lane = lax.iota(jnp.int32, 16)

        def slab(g, cur_ck):
            pltpu.sync_copy(eidx_hbm.at[pl.ds(s * gpt + g, 1)], idxv)
            cur, ck = cur_ck
            for b in range(KB):
                for k in range(CH // 16):
                    sl = pl.ds(16 * k, 16)
                    rv = idxv[0, b, sl]
                    cv = idxv[0, KB + b, sl]
                    off = cv - base
                    m = (off >= 0) & (off < half)
                    key = jnp.where(m, lane, 16 + lane)
                    _, rv_s = plsc.sort_key_val(key, rv)
                    _, off_s = plsc.sort_key_val(key, off)
                    srow[pl.ds(cur, 16)] = rv_s
                    soff[pl.ds(cur, 16)] = off_s
                    cur = cur + jnp.sum(m.astype(jnp.int32))
            for _ in range(3):
                cur, ck = lax.cond(cur >= CH, emit, keep, (cur, ck))
            return (cur, ck)

        cur, ck = lax.fori_loop(0, gpt, slab,
                                (jnp.int32(0), jnp.int32(0)))

        for i in range(CH // 16):
            srow[pl.ds(cur + i * 16, 16)] = jnp.zeros((16,), jnp.int32)
            soff[pl.ds(cur + i * 16, 16)] = jnp.full((16,), trash, jnp.int32)

        def emit_if(cur_ck):
            return lax.cond(cur_ck[0] > 0, emit, keep, cur_ck)
        for _ in range(4):
            cur, ck = emit_if((cur, ck))

        for i in range(CH // 16):
            srow[pl.ds(i * 16, 16)] = jnp.zeros((16,), jnp.int32)
            soff[pl.ds(i * 16, 16)] = jnp.full((16,), trash, jnp.int32)

        def emit_pad(ck_):
            o = (tbase + ck_) * 2 * CH
            pltpu.sync_copy(srow.at[pl.ds(0, CH)],
                            part_hbm.at[pl.ds(o, CH)])
            pltpu.sync_copy(soff.at[pl.ds(0, CH)],
                            part_hbm.at[pl.ds(o + CH, CH)])
            return ck_ + 1

        def keep1(ck_):
            return ck_
        for _ in range(6):
            ck = lax.cond((ck < 6) | (ck % 6 != 0), emit_pad, keep1, ck)

        lane = lax.iota(jnp.int32, 16)
        cbuf[pl.ds(0, 16)] = jnp.where(lane == 0, ck, 0)
        pltpu.sync_copy(cbuf, cnt_hbm.at[pl.ds(t * 16, 16)])

    return part_kernel


def _make_deg_kernel(n, half, aggr, capc):
    hpt = (-(-half // NS) + 7) // 8 * 8
    nch = -(-hpt // RB)
    mesh = plsc.VectorSubcoreMesh(core_axis_name="c", subcore_axis_name="s")

    @functools.partial(
        pl.kernel,
        out_type=jax.ShapeDtypeStruct((n, 16), jnp.float32),
        mesh=mesh,
        compiler_params=pltpu.CompilerParams(
            use_tc_tiling_on_sc=False, needs_layout_passes=False),
        scratch_types=[
            pltpu.VMEM_SHARED((aggr, 16), jnp.float32),
            pltpu.VMEM((RB, 16), jnp.float32),
            pltpu.VMEM((3, 2, CH), jnp.int32),
            pltpu.VMEM((16,), jnp.int32),
            pltpu.VMEM((RB, 16), jnp.float32),
        ],
    )
    def deg_kernel(part_hbm, cnt_hbm, deg_hbm, agg, fill, idxv, cbuf, stage):
        c = lax.axis_index("c")
        s = lax.axis_index("s")
        base = c * half
        t = c * NS + s
        tbase = t * capc

        def fill_rows(val):
            def frow(r, _):
                fill[r, pl.ds(0, 16)] = jnp.full((16,), val, jnp.float32)
                return 0
            lax.fori_loop(0, RB, frow, 0)

        fill_rows(0.0)

        def zchunk(j, _):
            lo = jnp.minimum(s * hpt + j * RB, aggr - RB)
            pltpu.sync_copy(fill, agg.at[pl.ds(lo, RB)])
            return 0
        lax.fori_loop(0, nch + 1, zchunk, 0)
        fill_rows(1.0)
        pltpu.sync_copy(cnt_hbm.at[pl.ds(t * 16, 16)], cbuf)
        nck = cbuf[pl.ds(0, 16)][0]
        plsc.subcore_barrier()

        def group(g, _):
            pltpu.sync_copy(part_hbm.at[pl.ds(tbase + g * 3, 3)], idxv)
            for b in range(3):
                pltpu.sync_copy(fill, agg.at[idxv.at[b, 1]], add=True)
            return 0
        lax.fori_loop(0, nck // 3, group, 0)
        plsc.subcore_barrier()

        def out_chunk(j, _):
            r = jnp.minimum(s * hpt + j * RB, half - RB)
            pltpu.sync_copy(agg.at[pl.ds(r, RB)], stage)
            pltpu.sync_copy(stage, deg_hbm.at[pl.ds(base + r, RB)])
            return 0
        lax.fori_loop(0, nch, out_chunk, 0)

    return deg_kernel


def _make_prop_kernel(n, half, aggr, capc):
    hpt = (-(-half // NS) + 7) // 8 * 8
    nch = -(-hpt // RB)
    mesh = plsc.VectorSubcoreMesh(core_axis_name="c", subcore_axis_name="s")

    @functools.partial(
        pl.kernel,
        out_type=jax.ShapeDtypeStruct((n, 32), jnp.float32),
        mesh=mesh,
        compiler_params=pltpu.CompilerParams(
            use_tc_tiling_on_sc=False, needs_layout_passes=False),
        scratch_types=[
            pltpu.VMEM_SHARED((aggr, 32), jnp.float32),
            pltpu.VMEM((3, 2, CH), jnp.int32),
            pltpu.VMEM((3, 2, CH), jnp.int32),
            pltpu.VMEM((3, CH, 32), jnp.float32),
            pltpu.VMEM((3, CH, 32), jnp.float32),
            pltpu.VMEM((16,), jnp.int32),
            pltpu.SemaphoreType.DMA,
            pltpu.SemaphoreType.DMA,
            pltpu.SemaphoreType.DMA,
            pltpu.SemaphoreType.DMA,
        ],
    )
    def prop_kernel(u_hbm, part_hbm, cnt_hbm, c32_hbm, g_hbm, out_hbm,
                    agg, idxv0, idxv1, rows0, rows1, cbuf, gsem0, gsem1,
                    ssem0, ssem1):
        c = lax.axis_index("c")
        s = lax.axis_index("s")
        base = c * half
        t = c * NS + s
        tbase = t * capc

        zbuf = rows0.at[0]
        bagg, bu = rows0.at[1], rows0.at[2]
        bc, bg = rows1.at[0], rows1.at[1]

        def zrow(r, _):
            zbuf[r, pl.ds(0, 16)] = jnp.zeros((16,), jnp.float32)
            zbuf[r, pl.ds(16, 16)] = jnp.zeros((16,), jnp.float32)
            return 0
        lax.fori_loop(0, RB, zrow, 0)

        def zchunk(j, _):
            lo = jnp.minimum(s * hpt + j * RB, aggr - RB)
            pltpu.sync_copy(zbuf, agg.at[pl.ds(lo, RB)])
            return 0
        lax.fori_loop(0, nch + 1, zchunk, 0)
        pltpu.sync_copy(cnt_hbm.at[pl.ds(t * 16, 16)], cbuf)
        nck = cbuf[pl.ds(0, 16)][0]
        plsc.subcore_barrier()

        def fire(gi, idxv, rows, gsem):
            pltpu.sync_copy(part_hbm.at[pl.ds(tbase + gi * 3, 3)], idxv)
            for b in range(3):
                pltpu.async_copy(u_hbm.at[idxv.at[b, 0]], rows.at[b], gsem)

        def gws(idxv, rows, gsem, ssem):
            for b in range(3):
                pltpu.make_async_copy(
                    u_hbm.at[idxv.at[b, 0]], rows.at[b], gsem).wait()
            for b in range(3):
                pltpu.async_copy(
                    rows.at[b], agg.at[idxv.at[b, 1]], ssem, add=True)

        def sw(idxv, rows, ssem):
            for b in range(3):
                pltpu.make_async_copy(
                    rows.at[b], agg.at[idxv.at[b, 1]], ssem).wait()

        fire(0, idxv0, rows0, gsem0)
        fire(1, idxv1, rows1, gsem1)

        def pair(j, _):
            gws(idxv0, rows0, gsem0, ssem0)
            gws(idxv1, rows1, gsem1, ssem1)
            sw(idxv0, rows0, ssem0)
            fire(2 * j + 2, idxv0, rows0, gsem0)
            sw(idxv1, rows1, ssem1)
            fire(2 * j + 3, idxv1, rows1, gsem1)
            return 0
        lax.fori_loop(0, nck // 6 - 1, pair, 0)
        gws(idxv0, rows0, gsem0, ssem0)
        gws(idxv1, rows1, gsem1, ssem1)
        sw(idxv0, rows0, ssem0)
        sw(idxv1, rows1, ssem1)
        plsc.subcore_barrier()

        def bchunk(j, _):
            r = jnp.minimum(s * hpt + j * RB, half - RB)
            lo = base + r
            pltpu.sync_copy(agg.at[pl.ds(r, RB)], bagg)
            pltpu.sync_copy(u_hbm.at[pl.ds(lo, RB)], bu)
            pltpu.sync_copy(c32_hbm.at[pl.ds(lo, RB)], bc)
            pltpu.sync_copy(g_hbm.at[pl.ds(lo, RB)], bg)

            def rrow(rr, _):
                for k in range(2):
                    sl = pl.ds(16 * k, 16)
                    bagg[rr, sl] = bc[rr, sl] * (bagg[rr, sl] + bu[rr, sl]) \
                        + bg[rr, sl]
                return 0
            lax.fori_loop(0, RB, rrow, 0)
            pltpu.sync_copy(bagg, out_hbm.at[pl.ds(lo, RB)])
            return 0
        lax.fori_loop(0, nch, bchunk, 0)

    return prop_kernel


def kernel(x, edge_index, W1, b1, W2, b2):
    n, ic = x.shape
    hc = W1.shape[1]
    oc = W2.shape[1]
    e = edge_index.shape[1]

    half = n // 2
    aggr = half + NS
    slab = KB * CH
    group_edges = NS * slab * 2
    ep = ((e + group_edges - 1) // group_edges) * group_edges
    nbs = ep // slab
    gpt = nbs // NS

    row = edge_index[0].astype(jnp.int32)
    col = edge_index[1].astype(jnp.int32)
    pad = ep - e
    row3 = jnp.concatenate(
        [row, jnp.zeros((pad,), jnp.int32)]).reshape(nbs, KB, CH)
    col3 = jnp.concatenate(
        [col, jnp.full((pad,), n, jnp.int32)]).reshape(nbs, KB, CH)
    eidx = jnp.concatenate([row3, col3], axis=1)

    grid = (n // BN,)
    h = pl.pallas_call(
        _mlp_body,
        grid=grid,
        in_specs=[
            pl.BlockSpec((BN, ic), lambda i: (i, 0)),
            pl.BlockSpec((ic, hc), lambda i: (0, 0)),
            pl.BlockSpec((1, hc), lambda i: (0, 0)),
            pl.BlockSpec((hc, oc), lambda i: (0, 0)),
            pl.BlockSpec((1, oc), lambda i: (0, 0)),
        ],
        out_specs=pl.BlockSpec((BN, oc), lambda i: (i, 0)),
        out_shape=jax.ShapeDtypeStruct((n, oc), jnp.float32),
    )(x, W1, b1.reshape(1, hc), W2, b2.reshape(1, oc))

    capc = (gpt * KB // 6 + 1) * 6
    part1, cnt1 = _make_part_kernel(n, half, gpt, capc)(eidx)
    part3 = part1.reshape(NC * NS * capc, 2, CH)

    deg16 = _make_deg_kernel(n, half, aggr, capc)(part3, cnt1)

    u, c32, g, sq = pl.pallas_call(
        _prep_body,
        grid=grid,
        in_specs=[
            pl.BlockSpec((BN, oc), lambda i: (i, 0)),
            pl.BlockSpec((BN, 16), lambda i: (i, 0)),
        ],
        out_specs=[
            pl.BlockSpec((BN, oc), lambda i: (i, 0)),
            pl.BlockSpec((BN, oc), lambda i: (i, 0)),
            pl.BlockSpec((BN, oc), lambda i: (i, 0)),
            pl.BlockSpec((BN, 16), lambda i: (i, 0)),
        ],
        out_shape=[
            jax.ShapeDtypeStruct((n, oc), jnp.float32),
            jax.ShapeDtypeStruct((n, oc), jnp.float32),
            jax.ShapeDtypeStruct((n, oc), jnp.float32),
            jax.ShapeDtypeStruct((n, 16), jnp.float32),
        ],
    )(h, deg16)

    prop = _make_prop_kernel(n, half, aggr, capc)
    for _ in range(K_PROP):
        u = prop(u, part3, cnt1, c32, g)

    out = pl.pallas_call(
        _final_body,
        grid=grid,
        in_specs=[
            pl.BlockSpec((BN, oc), lambda i: (i, 0)),
            pl.BlockSpec((BN, 16), lambda i: (i, 0)),
        ],
        out_specs=pl.BlockSpec((BN, oc), lambda i: (i, 0)),
        out_shape=jax.ShapeDtypeStruct((n, oc), jnp.float32),
    )(u, sq)
    return out
